# baseline (speedup 1.0000x reference)
import jax
import jax.numpy as jnp
from jax.experimental import pallas as pl
from jax.experimental.pallas import tpu as pltpu

_N = 1048576
_D = 2
_COLS = 128
_ROWS = (_N * _D) // _COLS
_BLK = 2048
_HALF = _COLS // 2


def _body(scale_ref, shift_ref, eps_ref, prob_ref, out_ref):
    s0, s1 = scale_ref[0], scale_ref[1]
    t0, t1 = shift_ref[0], shift_ref[1]
    e = eps_ref[...]
    lane = jax.lax.broadcasted_iota(jnp.int32, e.shape, 1)
    even = (lane & 1) == 0
    sv = jnp.where(even, s0, s1)
    tv = jnp.where(even, t0, t1)
    z = sv * e + tv
    u = z * z
    v = u + jnp.concatenate([u[:, 1:], u[:, :1]], axis=1)
    s2 = jnp.where(even, v, jnp.concatenate([v[:, -1:], v[:, :-1]], axis=1))
    c = jnp.float32(-0.5 * _D * jnp.log(2.0 * jnp.pi))
    lp = c - 0.5 * s2
    p_ = jnp.exp(lp - c)
    pr = prob_ref[...]
    w = jnp.concatenate([pr, pr], axis=1)
    pe = jnp.take_along_axis(w, lane >> 1, axis=1)
    out_ref[...] = jnp.where(p_ > pe, z, jnp.zeros_like(z))


def kernel(eps, prob, prop_scale, prop_shift):
    eps_flat = eps.reshape(_ROWS, _COLS)
    prob_flat = prob.reshape(_ROWS, _HALF)
    out = pl.pallas_call(
        _body,
        grid=(_ROWS // _BLK,),
        in_specs=[
            pl.BlockSpec(memory_space=pltpu.SMEM),
            pl.BlockSpec(memory_space=pltpu.SMEM),
            pl.BlockSpec((_BLK, _COLS), lambda i: (i, 0)),
            pl.BlockSpec((_BLK, _HALF), lambda i: (i, 0)),
        ],
        out_specs=pl.BlockSpec((_BLK, _COLS), lambda i: (i, 0)),
        out_shape=jax.ShapeDtypeStruct((_ROWS, _COLS), jnp.float32),
    )(prop_scale, prop_shift, eps_flat, prob_flat)
    return out.reshape(_N, _D)


# speedup vs baseline: 31.1807x; 31.1807x over previous
import functools
import math

import jax
import jax.numpy as jnp
from jax.experimental import pallas as pl
from jax.experimental.pallas import tpu as pltpu
from jax.experimental.pallas import tpu_sc as plsc

_N = 1048576
_D = 2
_LANES = 128
_G = _N // _LANES
_CG = 32
_SCL = 16
import numpy as _np

_C = float(_np.float32(-0.5 * _D * math.log(2.0 * math.pi)))


def _sc_call(eps_flat, prob_t, scale_mat, shift_mat):
    mesh = plsc.VectorSubcoreMesh(core_axis_name="c", subcore_axis_name="s")

    @functools.partial(
        pl.kernel,
        out_type=jax.ShapeDtypeStruct((_G * _D, _LANES), jnp.float32),
        mesh=mesh,
        scratch_types=[
            pltpu.VMEM((_D, _SCL), jnp.float32),
            pltpu.VMEM((_D, _SCL), jnp.float32),
        ],
    )
    def sck(eps_hbm, prob_hbm, scale_hbm, shift_hbm, out_hbm, sc_v, sh_v):
        pltpu.sync_copy(scale_hbm, sc_v)
        pltpu.sync_copy(shift_hbm, sh_v)
        s0, s1 = sc_v[0, :], sc_v[1, :]
        t0, t1 = sh_v[0, :], sh_v[1, :]

        def body(eps_vmem, prob_vmem, out_vmem):
            @pl.loop(0, _CG)
            def _(g):
                @pl.loop(0, _LANES, step=_SCL)
                def _(l):
                    sl = pl.ds(l, _SCL)
                    e0 = eps_vmem[2 * g, sl]
                    e1 = eps_vmem[2 * g + 1, sl]
                    z0 = s0 * e0 + t0
                    z1 = s1 * e1 + t1
                    s2 = z0 * z0 + z1 * z1
                    lp = _C - 0.5 * s2
                    p_ = jnp.exp(lp - _C)
                    acc = p_ > prob_vmem[g, sl]
                    zero = jnp.zeros((_SCL,), jnp.float32)
                    out_vmem[2 * g, sl] = jnp.where(acc, z0, zero)
                    out_vmem[2 * g + 1, sl] = jnp.where(acc, z1, zero)

        pltpu.emit_pipeline(
            body,
            grid=(_G // _CG,),
            in_specs=[
                pl.BlockSpec((_D * _CG, _LANES), lambda i: (i, 0)),
                pl.BlockSpec((_CG, _LANES), lambda i: (i, 0)),
            ],
            out_specs=[pl.BlockSpec((_D * _CG, _LANES), lambda i: (i, 0))],
            core_axis_name=("c", "s"),
            dimension_semantics=(pltpu.PARALLEL,),
        )(eps_hbm, prob_hbm, out_hbm)

    return sck(eps_flat, prob_t, scale_mat, shift_mat)


def kernel(eps, prob, prop_scale, prop_shift):
    eps_flat = (eps.reshape(_G, _LANES, _D)
                .transpose(0, 2, 1)
                .reshape(_G * _D, _LANES))
    prob_t = prob.reshape(_G, _LANES)
    scale_mat = jnp.broadcast_to(prop_scale[:, None], (_D, _SCL))
    shift_mat = jnp.broadcast_to(prop_shift[:, None], (_D, _SCL))
    out = _sc_call(eps_flat, prob_t, scale_mat, shift_mat)
    return (out.reshape(_G, _D, _LANES)
            .transpose(0, 2, 1)
            .reshape(_N, _D))


# speedup vs baseline: 55.8970x; 1.7927x over previous
import functools
import math

import jax
import jax.numpy as jnp
import numpy as _np
from jax.experimental import pallas as pl
from jax.experimental.pallas import tpu as pltpu
from jax.experimental.pallas import tpu_sc as plsc

_N = 1048576
_D = 2
_LANES = 128
_G = _N // _LANES
_GS = 1024
_GT = _G - _GS
_BLK = 1024
_CG = 8
_SCL = 16
_C = float(_np.float32(-0.5 * _D * math.log(2.0 * math.pi)))


def _tc_body(scale_ref, shift_ref, eps_ref, prob_ref, out_ref):
    e = eps_ref[...]
    sub = jax.lax.broadcasted_iota(jnp.int32, e.shape, 0)
    even = (sub & 1) == 0
    sv = jnp.where(even, scale_ref[0], scale_ref[1])
    tv = jnp.where(even, shift_ref[0], shift_ref[1])
    z = sv * e + tv
    u = z * z
    v = u + jnp.concatenate([u[1:, :], u[:1, :]], axis=0)
    s2 = jnp.where(even, v, jnp.concatenate([v[-1:, :], v[:-1, :]], axis=0))
    lp = _C - 0.5 * s2
    p_ = jnp.exp(lp - _C)
    pe = jnp.repeat(prob_ref[...], _D, axis=0)
    out_ref[...] = jnp.where(p_ > pe, z, jnp.zeros_like(z))


def _tc_call(eps_flat, prob_t, prop_scale, prop_shift):
    return pl.pallas_call(
        _tc_body,
        grid=(_GT // _BLK,),
        in_specs=[
            pl.BlockSpec(memory_space=pltpu.SMEM),
            pl.BlockSpec(memory_space=pltpu.SMEM),
            pl.BlockSpec((_BLK * _D, _LANES), lambda i: (i, 0)),
            pl.BlockSpec((_BLK, _LANES), lambda i: (i, 0)),
        ],
        out_specs=pl.BlockSpec((_BLK * _D, _LANES), lambda i: (i, 0)),
        out_shape=jax.ShapeDtypeStruct((_GT * _D, _LANES), jnp.float32),
    )(prop_scale, prop_shift, eps_flat, prob_t)


def _sc_call(eps_flat, prob_t, scale_mat, shift_mat):
    mesh = plsc.VectorSubcoreMesh(core_axis_name="c", subcore_axis_name="s")
    off = _GT // _CG

    @functools.partial(
        pl.kernel,
        out_type=jax.ShapeDtypeStruct((_GS * _D, _LANES), jnp.float32),
        mesh=mesh,
        scratch_types=[
            pltpu.VMEM((_D, _SCL), jnp.float32),
            pltpu.VMEM((_D, _SCL), jnp.float32),
        ],
    )
    def sck(eps_hbm, prob_hbm, scale_hbm, shift_hbm, out_hbm, sc_v, sh_v):
        pltpu.sync_copy(scale_hbm, sc_v)
        pltpu.sync_copy(shift_hbm, sh_v)
        s0, s1 = sc_v[0, :], sc_v[1, :]
        t0, t1 = sh_v[0, :], sh_v[1, :]

        def body(eps_vmem, prob_vmem, out_vmem):
            @pl.loop(0, _CG)
            def _(g):
                @pl.loop(0, _LANES, step=_SCL)
                def _(l):
                    sl = pl.ds(l, _SCL)
                    e0 = eps_vmem[2 * g, sl]
                    e1 = eps_vmem[2 * g + 1, sl]
                    z0 = s0 * e0 + t0
                    z1 = s1 * e1 + t1
                    s2 = z0 * z0 + z1 * z1
                    lp = _C - 0.5 * s2
                    p_ = jnp.exp(lp - _C)
                    acc = p_ > prob_vmem[g, sl]
                    zero = jnp.zeros((_SCL,), jnp.float32)
                    out_vmem[2 * g, sl] = jnp.where(acc, z0, zero)
                    out_vmem[2 * g + 1, sl] = jnp.where(acc, z1, zero)

        pltpu.emit_pipeline(
            body,
            grid=(_GS // _CG,),
            in_specs=[
                pl.BlockSpec((_D * _CG, _LANES), lambda i: (i + off, 0)),
                pl.BlockSpec((_CG, _LANES), lambda i: (i + off, 0)),
            ],
            out_specs=[pl.BlockSpec((_D * _CG, _LANES), lambda i: (i, 0))],
            core_axis_name=("c", "s"),
            dimension_semantics=(pltpu.PARALLEL,),
        )(eps_hbm, prob_hbm, out_hbm)

    return sck(eps_flat, prob_t, scale_mat, shift_mat)


def kernel(eps, prob, prop_scale, prop_shift):
    eps_flat = (eps.reshape(_G, _LANES, _D)
                .transpose(0, 2, 1)
                .reshape(_G * _D, _LANES))
    prob_t = prob.reshape(_G, _LANES)
    scale_mat = jnp.broadcast_to(prop_scale[:, None], (_D, _SCL))
    shift_mat = jnp.broadcast_to(prop_shift[:, None], (_D, _SCL))
    out_sc = _sc_call(eps_flat, prob_t, scale_mat, shift_mat)
    out_tc = _tc_call(eps_flat, prob_t, prop_scale, prop_shift)
    out = jnp.concatenate([out_tc, out_sc], axis=0)
    return (out.reshape(_G, _D, _LANES)
            .transpose(0, 2, 1)
            .reshape(_N, _D))


# speedup vs baseline: 141.6041x; 2.5333x over previous
import jax
import jax.numpy as jnp
from jax.experimental import pallas as pl
from jax.experimental.pallas import tpu as pltpu

_N = 1048576
_D = 2
_LANES = 128
_G = _N // _LANES
_BLK = 1024


def _body(scale_ref, shift_ref, eps_ref, prob_ref, out_ref):
    e = eps_ref[...]
    sub = jax.lax.broadcasted_iota(jnp.int32, e.shape, 0)
    even = (sub & 1) == 0
    sv = jnp.where(even, scale_ref[0], scale_ref[1])
    tv = jnp.where(even, shift_ref[0], shift_ref[1])
    z = sv * e + tv
    u = z * z
    v = u + jnp.concatenate([u[1:, :], u[:1, :]], axis=0)
    s2 = jnp.where(even, v, jnp.concatenate([v[-1:, :], v[:-1, :]], axis=0))
    c = jnp.float32(-0.5 * _D * jnp.log(2.0 * jnp.pi))
    lp = c - 0.5 * s2
    p_ = jnp.exp(lp - c)
    pe = jnp.repeat(prob_ref[...], _D, axis=0)
    out_ref[...] = jnp.where(p_ > pe, z, jnp.zeros_like(z))


def kernel(eps, prob, prop_scale, prop_shift):
    eps_flat = (eps.reshape(_G, _LANES, _D)
                .transpose(0, 2, 1)
                .reshape(_G * _D, _LANES))
    prob_t = prob.reshape(_G, _LANES)
    out = pl.pallas_call(
        _body,
        grid=(_G // _BLK,),
        in_specs=[
            pl.BlockSpec(memory_space=pltpu.SMEM),
            pl.BlockSpec(memory_space=pltpu.SMEM),
            pl.BlockSpec((_BLK * _D, _LANES), lambda i: (i, 0)),
            pl.BlockSpec((_BLK, _LANES), lambda i: (i, 0)),
        ],
        out_specs=pl.BlockSpec((_BLK * _D, _LANES), lambda i: (i, 0)),
        out_shape=jax.ShapeDtypeStruct((_G * _D, _LANES), jnp.float32),
    )(prop_scale, prop_shift, eps_flat, prob_t)
    return (out.reshape(_G, _D, _LANES)
            .transpose(0, 2, 1)
            .reshape(_N, _D))
